# Initial kernel scaffold; baseline (speedup 1.0000x reference)
import jax
import jax.numpy as jnp
from jax.experimental import pallas as pl


def kernel(visitorid, event, weight, timestamp, user_table, action_table, weight_table, time_table, boundaries, time_mean, time_std):
    raise NotImplementedError("write your pallas kernel here")


# speedup vs baseline: 1.5258x; 1.5258x over previous
import functools

import jax
import jax.numpy as jnp
from jax import lax
from jax.experimental import pallas as pl
from jax.experimental.pallas import tpu as pltpu
from jax.experimental.pallas import tpu_sc as plsc

B = 16384
D = 64
OUT_D = 3 * D + 1 + D
NC = 2
NS = 16
NW = NC * NS
RPW = B // NW
CHUNK = 128
NCHUNK = RPW // CHUNK
HALF = 256
LANES = 16

A_OFF = 0
W_OFF = 5
T_OFF = 9
NBND = 120


def _bucketize(bnd_ref, ts16):
    pos = jnp.zeros((LANES,), jnp.int32)
    for step in (64, 32, 16, 8, 4, 2, 1):
        cand = pos + step
        safe = jnp.minimum(cand, NBND) - 1
        v = plsc.load_gather(bnd_ref, [safe])
        ok = (cand <= NBND) & (v < ts16)
        pos = jnp.where(ok, cand, pos)
    return pos


def _body(viz_hbm, ev_hbm, wt_hbm, ts_hbm, utab_hbm, comb_hbm, bnd_hbm,
          tm_hbm, td_hbm, out_hbm,
          idx_v, uv, ob, ev_v, wt_v, ts_v, comb_v, bnd_v, tm_v, td_v, gsem):
    cid = lax.axis_index("c")
    sid = lax.axis_index("s")
    wid = sid * NC + cid
    base = wid * RPW

    for c in range(NCHUNK):
        pltpu.sync_copy(viz_hbm.at[pl.ds(base + c * CHUNK, CHUNK)],
                        idx_v.at[c])
    handles = [
        pltpu.async_copy(utab_hbm.at[idx_v.at[c]],
                         uv.at[pl.ds(c * CHUNK, CHUNK)], gsem)
        for c in range(NCHUNK)
    ]
    pltpu.sync_copy(ev_hbm.at[pl.ds(base, RPW)], ev_v)
    pltpu.sync_copy(wt_hbm.at[pl.ds(base, RPW)], wt_v)
    pltpu.sync_copy(ts_hbm.at[pl.ds(base, RPW)], ts_v)
    pltpu.sync_copy(comb_hbm, comb_v)
    pltpu.sync_copy(bnd_hbm, bnd_v)
    pltpu.sync_copy(tm_hbm, tm_v)
    pltpu.sync_copy(td_hbm, td_v)

    tm16 = tm_v[...]
    td16 = td_v[...]
    lane = lax.iota(jnp.int32, LANES)

    for h in handles:
        h.wait()

    for half in range(RPW // HALF):
        def group(g, carry, half=half):
            row_loc = g * LANES + lane
            srow = half * HALF + g * LANES + lane
            ev16 = ev_v[pl.ds(half * HALF + g * LANES, LANES)]
            wt16 = wt_v[pl.ds(half * HALF + g * LANES, LANES)]
            ts16 = ts_v[pl.ds(half * HALF + g * LANES, LANES)]
            bk16 = _bucketize(bnd_v, ts16)
            arow = ev16 + A_OFF
            wrow = wt16 + W_OFF
            trow = bk16 + T_OFF
            cont = (ts16 - tm16) / td16
            plsc.store_scatter(ob, [row_loc,
                                    jnp.full((LANES,), 3 * D, jnp.int32)],
                               cont)
            for j in range(D):
                colj = jnp.full((LANES,), j, jnp.int32)
                uvv = plsc.load_gather(uv, [srow, colj])
                plsc.store_scatter(ob, [row_loc, colj], uvv)
                av = plsc.load_gather(comb_v, [arow, colj])
                plsc.store_scatter(
                    ob, [row_loc, jnp.full((LANES,), D + j, jnp.int32)], av)
                wv = plsc.load_gather(comb_v, [wrow, colj])
                plsc.store_scatter(
                    ob, [row_loc, jnp.full((LANES,), 2 * D + j, jnp.int32)],
                    wv)
                tv = plsc.load_gather(comb_v, [trow, colj])
                plsc.store_scatter(
                    ob, [row_loc, jnp.full((LANES,), 3 * D + 1 + j, jnp.int32)],
                    tv)
            return carry

        lax.fori_loop(0, HALF // LANES, group, 0)
        pltpu.sync_copy(ob, out_hbm.at[pl.ds(base + half * HALF, HALF)])


_sc_call = functools.partial(
    pl.kernel,
    out_type=jax.ShapeDtypeStruct((B, OUT_D), jnp.float32),
    mesh=plsc.VectorSubcoreMesh(core_axis_name="c", subcore_axis_name="s"),
    compiler_params=pltpu.CompilerParams(
        needs_layout_passes=False, use_tc_tiling_on_sc=False),
    scratch_types=[
        pltpu.VMEM((NCHUNK, CHUNK), jnp.int32),
        pltpu.VMEM((RPW, D), jnp.float32),
        pltpu.VMEM((HALF, OUT_D), jnp.float32),
        pltpu.VMEM((RPW,), jnp.int32),
        pltpu.VMEM((RPW,), jnp.int32),
        pltpu.VMEM((RPW,), jnp.float32),
        pltpu.VMEM((130, D), jnp.float32),
        pltpu.VMEM((128,), jnp.float32),
        pltpu.VMEM((LANES,), jnp.float32),
        pltpu.VMEM((LANES,), jnp.float32),
        pltpu.SemaphoreType.DMA,
    ],
)(_body)


def kernel(visitorid, event, weight, timestamp, user_table, action_table,
           weight_table, time_table, boundaries, time_mean, time_std):
    viz = visitorid.astype(jnp.int32)
    ev = event.astype(jnp.int32)
    wt = weight.astype(jnp.int32)
    ts = timestamp.astype(jnp.float32)
    comb = jnp.concatenate([action_table, weight_table, time_table], axis=0)
    bnd = jnp.concatenate(
        [boundaries.astype(jnp.float32), jnp.zeros((8,), jnp.float32)])
    tm = jnp.full((LANES,), time_mean, jnp.float32)
    td = jnp.full((LANES,), time_std, jnp.float32)
    return _sc_call(viz, ev, wt, ts, user_table, comb, bnd, tm, td)
